# baseline (speedup 1.0000x reference)
import functools

import jax
import jax.numpy as jnp
from jax import lax
from jax.experimental import pallas as pl
from jax.experimental.pallas import tpu as pltpu
from jax.experimental.pallas import tpu_sc as plsc

NUM_HEADS = 16
HEAD_DIM = 128
HIDDEN = 2048
NUM_HASHES = 8
SCALE = HEAD_DIM ** (-0.5)
B, S = 2, 4096
GROUP = 8
KV = 2 * HEAD_DIM
CHUNK = 128
N_CHUNKS = S // CHUNK


def _q_body(x_ref, w_ref, b_ref, projt_ref, q_ref, bk_ref):
    acc = jnp.dot(x_ref[...], w_ref[...], preferred_element_type=jnp.float32)
    acc = acc + b_ref[...]
    bm = acc.shape[0]
    heads = acc.shape[1] // HEAD_DIM
    q_ref[...] = acc.reshape(bm, heads, HEAD_DIM).swapaxes(0, 1)[None]
    powers = (2 ** lax.iota(jnp.int32, NUM_HASHES))[None, :]
    for hh in range(heads):
        qp = jnp.dot(acc[:, hh * HEAD_DIM:(hh + 1) * HEAD_DIM],
                     projt_ref[...], preferred_element_type=jnp.float32)
        bits = (qp > 0).astype(jnp.int32)
        bk_ref[0, hh, :] = jnp.sum(bits * powers, axis=-1)


def _q_proj(x2d, wq, bq, proj, bm, bn):
    m_tiles = x2d.shape[0] // bm
    n_tiles = HIDDEN // bn
    heads_per_n = bn // HEAD_DIM
    s_tiles = S // bm
    return pl.pallas_call(
        _q_body,
        grid=(n_tiles, m_tiles),
        in_specs=[
            pl.BlockSpec((bm, HIDDEN), lambda n, m: (m, 0)),
            pl.BlockSpec((HIDDEN, bn), lambda n, m: (0, n)),
            pl.BlockSpec((1, bn), lambda n, m: (0, n)),
            pl.BlockSpec((HEAD_DIM, NUM_HASHES), lambda n, m: (0, 0)),
        ],
        out_specs=[
            pl.BlockSpec((1, heads_per_n, bm, HEAD_DIM),
                         lambda n, m: (m // s_tiles, n, m % s_tiles, 0)),
            pl.BlockSpec((1, heads_per_n, bm), lambda n, m: (n, 0, m)),
        ],
        out_shape=[
            jax.ShapeDtypeStruct((B, NUM_HEADS, S, HEAD_DIM), jnp.float32),
            jax.ShapeDtypeStruct((n_tiles, heads_per_n, B * S), jnp.int32),
        ],
    )(x2d, wq, bq.reshape(1, -1), proj.T)


def _kv_body(x_ref, w_ref, b_ref, out_ref):
    acc = jnp.dot(x_ref[...], w_ref[...], preferred_element_type=jnp.float32)
    acc = acc + b_ref[...]
    bm = acc.shape[0]
    out_ref[...] = acc.reshape(bm, -1, KV).swapaxes(0, 1)[None]


def _kv_proj(xb, wkv, bkv, bm, bn):
    m_tiles = xb.shape[0] // bm
    n_tiles = wkv.shape[1] // bn
    heads_per_n = bn // KV
    s_tiles = S // bm
    return pl.pallas_call(
        _kv_body,
        grid=(n_tiles, m_tiles),
        in_specs=[
            pl.BlockSpec((bm, HIDDEN), lambda n, m: (m, 0)),
            pl.BlockSpec((HIDDEN, bn), lambda n, m: (0, n)),
            pl.BlockSpec((1, bn), lambda n, m: (0, n)),
        ],
        out_specs=pl.BlockSpec(
            (1, heads_per_n, bm, KV),
            lambda n, m: (m // s_tiles, n, m % s_tiles, 0)),
        out_shape=jax.ShapeDtypeStruct((B, NUM_HEADS, S, KV), jnp.float32),
    )(xb, wkv, bkv.reshape(1, -1))


def _sc_sort_gather(buckets_flat, q_table, kv_table):
    mesh = plsc.VectorSubcoreMesh(core_axis_name="c", subcore_axis_name="s")

    @functools.partial(
        pl.kernel,
        out_type=[
            jax.ShapeDtypeStruct((B * S * NUM_HEADS, HEAD_DIM), jnp.float32),
            jax.ShapeDtypeStruct((B * S * NUM_HEADS, KV), jnp.float32),
        ],
        mesh=mesh,
        scratch_types=[
            pltpu.VMEM((S,), jnp.int32),
            pltpu.VMEM((16 * 256,), jnp.int32),
            pltpu.VMEM((S,), jnp.int32),
            pltpu.VMEM((N_CHUNKS, CHUNK), jnp.int32),
            pltpu.VMEM((CHUNK, HEAD_DIM), jnp.float32),
            pltpu.VMEM((CHUNK, HEAD_DIM), jnp.float32),
            pltpu.VMEM((CHUNK, KV), jnp.float32),
            pltpu.VMEM((CHUNK, KV), jnp.float32),
            pltpu.SMEM((256,), jnp.int32),
            pltpu.SemaphoreType.DMA,
            pltpu.SemaphoreType.DMA,
            pltpu.SemaphoreType.DMA,
            pltpu.SemaphoreType.DMA,
            pltpu.SemaphoreType.DMA,
            pltpu.SemaphoreType.DMA,
            pltpu.SemaphoreType.DMA,
            pltpu.SemaphoreType.DMA,
        ],
        compiler_params=pltpu.CompilerParams(needs_layout_passes=False),
    )
    def sc_kernel(buckets_hbm, q_hbm, kv_hbm, qs_hbm, kvs_hbm,
                  keys, hist2d, gidx, sidx, qr0, qr1, kvr0, kvr1, offs,
                  gq0, gq1, gkv0, gkv1, sq0, sq1, skv0, skv1):
        w = lax.axis_index("s") * 2 + lax.axis_index("c")
        b = w // NUM_HEADS
        h = w % NUM_HEADS
        src_base = w * S
        dst_base = b * S * NUM_HEADS + h

        lane = lax.iota(jnp.int32, 16)
        zero16 = jnp.zeros((16,), jnp.int32)
        ones16 = jnp.ones((16,), jnp.int32)

        pltpu.sync_copy(buckets_hbm.at[pl.ds(w * S, S)], keys)

        for j in range(16 * 256 // 16):
            hist2d[pl.ds(j * 16, 16)] = zero16

        lane256 = lane * 256
        def hist_body(c, carry):
            k16 = keys[pl.ds(c * 16, 16)]
            slot = lane256 + k16
            cnt = plsc.load_gather(hist2d, [slot])
            plsc.store_scatter(hist2d, [slot], cnt + ones16)
            return carry
        lax.fori_loop(0, S // 16, hist_body, 0, unroll=4)

        carry_in = jnp.int32(0)
        for g in range(16):
            tot = zero16
            for l in range(16):
                tot = tot + hist2d[pl.ds(l * 256 + g * 16, 16)]
            incl = plsc.cumsum(tot)
            excl = incl - tot + carry_in
            for l in range(16):
                offs[g * 16 + l] = excl[l]
            carry_in = carry_in + incl[15]

        def place_body(c, carry):
            k16 = keys[pl.ds(c * 16, 16)]
            src16 = src_base + c * 16 + lane
            rvec = zero16
            for l in range(16):
                k = k16[l]
                r = offs[k]
                offs[k] = r + 1
                rvec = jnp.where(lane == l, r, rvec)
            plsc.store_scatter(gidx, [rvec], src16)
            return carry
        lax.fori_loop(0, S // 16, place_body, 0)

        for c in range(N_CHUNKS):
            for g in range(CHUNK // 16):
                t0 = c * CHUNK + g * 16
                sidx[c, pl.ds(g * 16, 16)] = (
                    dst_base + (t0 + lane) * NUM_HEADS)

        qbufs = (qr0, qr1)
        kvbufs = (kvr0, kvr1)
        gqs = (gq0, gq1)
        gkvs = (gkv0, gkv1)
        sqs = (sq0, sq1)
        skvs = (skv0, skv1)

        def chunk_step(c, p):
            @pl.when(c >= 2)
            def _():
                pltpu.make_async_copy(qbufs[p], qs_hbm.at[sidx.at[c - 2]],
                                      sqs[p]).wait()
                pltpu.make_async_copy(kvbufs[p], kvs_hbm.at[sidx.at[c - 2]],
                                      skvs[p]).wait()

            gi = gidx.at[pl.ds(c * CHUNK, CHUNK)]
            pltpu.make_async_copy(q_hbm.at[gi], qbufs[p], gqs[p]).start()
            pltpu.make_async_copy(kv_hbm.at[gi], kvbufs[p], gkvs[p]).start()
            pltpu.make_async_copy(q_hbm.at[gi], qbufs[p], gqs[p]).wait()
            pltpu.make_async_copy(kv_hbm.at[gi], kvbufs[p], gkvs[p]).wait()
            pltpu.make_async_copy(qbufs[p], qs_hbm.at[sidx.at[c]],
                                  sqs[p]).start()
            pltpu.make_async_copy(kvbufs[p], kvs_hbm.at[sidx.at[c]],
                                  skvs[p]).start()

        def outer(c, carry):
            chunk_step(c * 2, 0)
            chunk_step(c * 2 + 1, 1)
            return carry
        lax.fori_loop(0, N_CHUNKS // 2, outer, 0)

        for p in range(2):
            c = N_CHUNKS - 2 + p
            pltpu.make_async_copy(qbufs[p], qs_hbm.at[sidx.at[c]],
                                  sqs[p]).wait()
            pltpu.make_async_copy(kvbufs[p], kvs_hbm.at[sidx.at[c]],
                                  skvs[p]).wait()

    return sc_kernel(buckets_flat, q_table, kv_table)


def _attn_body(q_ref, kv_ref, wo_ref, bo_ref, out_ref, att_ref):
    n_groups = q_ref.shape[0]

    r = lax.broadcasted_iota(jnp.int32, (GROUP * NUM_HEADS,
                                         GROUP * NUM_HEADS), 0)
    c = lax.broadcasted_iota(jnp.int32, (GROUP * NUM_HEADS,
                                         GROUP * NUM_HEADS), 1)
    same_pos = (r // NUM_HEADS) == (c // NUM_HEADS)

    for g in range(n_groups):
        qg = q_ref[g].astype(jnp.bfloat16)
        kvg = kv_ref[g].astype(jnp.bfloat16)
        kg = kvg[:, :HEAD_DIM]
        vg = kvg[:, HEAD_DIM:]
        s = lax.dot_general(
            qg, kg, (((1,), (1,)), ((), ())),
            preferred_element_type=jnp.float32) * SCALE
        s = jnp.where(same_pos, s, -jnp.inf)
        s = s - jnp.max(s, axis=-1, keepdims=True)
        e = jnp.exp(s)
        p = (e / jnp.sum(e, axis=-1, keepdims=True)).astype(jnp.bfloat16)
        og = jnp.dot(p, vg, preferred_element_type=jnp.float32)
        att_ref[g * GROUP:(g + 1) * GROUP, :] = og.astype(
            jnp.bfloat16).reshape(GROUP, HIDDEN)

    out_ref[...] = (
        jnp.dot(att_ref[...], wo_ref[...], preferred_element_type=jnp.float32)
        + bo_ref[...])


def _attn_proj(q_s, kv_s, wo_bf16, bo, bm):
    m_tiles = (B * S) // bm
    g = bm // GROUP
    return pl.pallas_call(
        _attn_body,
        grid=(m_tiles,),
        in_specs=[
            pl.BlockSpec((g, GROUP * NUM_HEADS, HEAD_DIM),
                         lambda m: (m, 0, 0)),
            pl.BlockSpec((g, GROUP * NUM_HEADS, KV), lambda m: (m, 0, 0)),
            pl.BlockSpec((HIDDEN, HIDDEN), lambda m: (0, 0)),
            pl.BlockSpec((1, HIDDEN), lambda m: (0, 0)),
        ],
        out_specs=pl.BlockSpec((bm, HIDDEN), lambda m: (m, 0)),
        out_shape=jax.ShapeDtypeStruct((B * S, HIDDEN), jnp.float32),
        scratch_shapes=[pltpu.VMEM((bm, HIDDEN), jnp.bfloat16)],
    )(q_s, kv_s, wo_bf16, bo.reshape(1, -1))


@jax.jit
def kernel(x, Wq, bq, Wk, bk, Wv, bv, Wo, bo, proj):
    x2d = x.reshape(B * S, HIDDEN)
    xb = x2d.astype(jnp.bfloat16)
    wkv = jnp.concatenate(
        [Wk.reshape(HIDDEN, NUM_HEADS, 1, HEAD_DIM),
         Wv.reshape(HIDDEN, NUM_HEADS, 1, HEAD_DIM)], axis=2,
    ).reshape(HIDDEN, 2 * HIDDEN).astype(jnp.bfloat16)
    bkv = jnp.concatenate(
        [bk.reshape(NUM_HEADS, 1, HEAD_DIM), bv.reshape(NUM_HEADS, 1,
                                                        HEAD_DIM)], axis=1,
    ).reshape(2 * HIDDEN)

    q_tab, bk6 = _q_proj(x2d, Wq, bq, proj, bm=512, bn=1024)
    kv_tab = _kv_proj(xb, wkv, bkv, bm=512, bn=2048)

    buckets = bk6.reshape(NUM_HEADS, B, S).swapaxes(0, 1).reshape(-1)

    q_s, kv_s = _sc_sort_gather(
        buckets,
        q_tab.reshape(B * NUM_HEADS * S, HEAD_DIM),
        kv_tab.reshape(B * NUM_HEADS * S, KV))
    q_s = q_s.reshape(B * S // GROUP, GROUP * NUM_HEADS, HEAD_DIM)
    kv_s = kv_s.reshape(B * S // GROUP, GROUP * NUM_HEADS, KV)

    out = _attn_proj(q_s, kv_s, Wo.astype(jnp.bfloat16), bo, bm=256)
    return out.reshape(B, S, HIDDEN)


# speedup vs baseline: 1.0620x; 1.0620x over previous
import functools

import jax
import jax.numpy as jnp
from jax import lax
from jax.experimental import pallas as pl
from jax.experimental.pallas import tpu as pltpu
from jax.experimental.pallas import tpu_sc as plsc

NUM_HEADS = 16
HEAD_DIM = 128
HIDDEN = 2048
NUM_HASHES = 8
SCALE = HEAD_DIM ** (-0.5)
B, S = 2, 4096
GROUP = 8
KV = 2 * HEAD_DIM
QW = HEAD_DIM // 2
KVW = HEAD_DIM
CHUNK = 128
N_CHUNKS = S // CHUNK
MASK_HI = -65536


def _rne_bf16_bits(x_f32):
    u = pltpu.bitcast(x_f32, jnp.int32)
    return u + 0x7FFF + jnp.bitwise_and(lax.shift_right_logical(u, 16), 1)


def _q_body(x_ref, w_ref, b_ref, projt_ref, q_ref, bk_ref):
    acc = jnp.dot(x_ref[...], w_ref[...], preferred_element_type=jnp.float32)
    acc = acc + b_ref[...]
    bm = acc.shape[0]
    heads = acc.shape[1] // HEAD_DIM
    q_ref[...] = acc.reshape(bm, heads, HEAD_DIM).swapaxes(0, 1)[None]
    powers = (2 ** lax.iota(jnp.int32, NUM_HASHES))[None, :]
    for hh in range(heads):
        qp = jnp.dot(acc[:, hh * HEAD_DIM:(hh + 1) * HEAD_DIM],
                     projt_ref[...], preferred_element_type=jnp.float32)
        bits = (qp > 0).astype(jnp.int32)
        bk_ref[0, hh, :] = jnp.sum(bits * powers, axis=-1)


def _q_proj(x2d, wq, bq, proj, bm, bn):
    m_tiles = x2d.shape[0] // bm
    n_tiles = HIDDEN // bn
    heads_per_n = bn // HEAD_DIM
    s_tiles = S // bm
    return pl.pallas_call(
        _q_body,
        grid=(n_tiles, m_tiles),
        in_specs=[
            pl.BlockSpec((bm, HIDDEN), lambda n, m: (m, 0)),
            pl.BlockSpec((HIDDEN, bn), lambda n, m: (0, n)),
            pl.BlockSpec((1, bn), lambda n, m: (0, n)),
            pl.BlockSpec((HEAD_DIM, NUM_HASHES), lambda n, m: (0, 0)),
        ],
        out_specs=[
            pl.BlockSpec((1, heads_per_n, bm, HEAD_DIM),
                         lambda n, m: (m // s_tiles, n, m % s_tiles, 0)),
            pl.BlockSpec((1, heads_per_n, bm), lambda n, m: (n, 0, m)),
        ],
        out_shape=[
            jax.ShapeDtypeStruct((B, NUM_HEADS, S, HEAD_DIM), jnp.float32),
            jax.ShapeDtypeStruct((n_tiles, heads_per_n, B * S), jnp.int32),
        ],
    )(x2d, wq, bq.reshape(1, -1), proj.T)


def _kv_body(x_ref, w_ref, b_ref, out_ref):
    acc = jnp.dot(x_ref[...], w_ref[...], preferred_element_type=jnp.float32)
    acc = acc + b_ref[...]
    heads = acc.shape[1] // KV
    for hh in range(heads):
        rk = _rne_bf16_bits(acc[:, hh * KV:hh * KV + HEAD_DIM])
        rv = _rne_bf16_bits(acc[:, hh * KV + HEAD_DIM:(hh + 1) * KV])
        out_ref[0, hh] = jnp.bitwise_or(
            lax.shift_right_logical(rk, 16), jnp.bitwise_and(rv, MASK_HI))


def _kv_proj(xb, wkv, bkv, bm, bn):
    m_tiles = xb.shape[0] // bm
    n_tiles = wkv.shape[1] // bn
    heads_per_n = bn // KV
    s_tiles = S // bm
    return pl.pallas_call(
        _kv_body,
        grid=(n_tiles, m_tiles),
        in_specs=[
            pl.BlockSpec((bm, HIDDEN), lambda n, m: (m, 0)),
            pl.BlockSpec((HIDDEN, bn), lambda n, m: (0, n)),
            pl.BlockSpec((1, bn), lambda n, m: (0, n)),
        ],
        out_specs=pl.BlockSpec(
            (1, heads_per_n, bm, KVW),
            lambda n, m: (m // s_tiles, n, m % s_tiles, 0)),
        out_shape=jax.ShapeDtypeStruct((B, NUM_HEADS, S, KVW), jnp.int32),
    )(xb, wkv, bkv.reshape(1, -1))


def _sc_sort_gather(buckets_flat, q_table, kv_table):
    mesh = plsc.VectorSubcoreMesh(core_axis_name="c", subcore_axis_name="s")

    @functools.partial(
        pl.kernel,
        out_type=[
            jax.ShapeDtypeStruct((B * S * NUM_HEADS, HEAD_DIM), jnp.float32),
            jax.ShapeDtypeStruct((B * S * NUM_HEADS, KVW), jnp.int32),
        ],
        mesh=mesh,
        scratch_types=[
            pltpu.VMEM((S,), jnp.int32),
            pltpu.VMEM((16 * 256,), jnp.int32),
            pltpu.VMEM((S,), jnp.int32),
            pltpu.VMEM((N_CHUNKS, CHUNK), jnp.int32),
            pltpu.VMEM((CHUNK, HEAD_DIM), jnp.float32),
            pltpu.VMEM((CHUNK, HEAD_DIM), jnp.float32),
            pltpu.VMEM((CHUNK, KVW), jnp.int32),
            pltpu.VMEM((CHUNK, KVW), jnp.int32),
            pltpu.SMEM((256,), jnp.int32),
            pltpu.SemaphoreType.DMA,
            pltpu.SemaphoreType.DMA,
            pltpu.SemaphoreType.DMA,
            pltpu.SemaphoreType.DMA,
            pltpu.SemaphoreType.DMA,
            pltpu.SemaphoreType.DMA,
            pltpu.SemaphoreType.DMA,
            pltpu.SemaphoreType.DMA,
        ],
        compiler_params=pltpu.CompilerParams(needs_layout_passes=False),
    )
    def sc_kernel(buckets_hbm, q_hbm, kv_hbm, qs_hbm, kvs_hbm,
                  keys, hist2d, gidx, sidx, qr0, qr1, kvr0, kvr1, offs,
                  gq0, gq1, gkv0, gkv1, sq0, sq1, skv0, skv1):
        w = lax.axis_index("s") * 2 + lax.axis_index("c")
        b = w // NUM_HEADS
        h = w % NUM_HEADS
        src_base = w * S
        dst_base = b * S * NUM_HEADS + h

        lane = lax.iota(jnp.int32, 16)
        zero16 = jnp.zeros((16,), jnp.int32)
        ones16 = jnp.ones((16,), jnp.int32)

        pltpu.sync_copy(buckets_hbm.at[pl.ds(w * S, S)], keys)

        for j in range(16 * 256 // 16):
            hist2d[pl.ds(j * 16, 16)] = zero16

        lane256 = lane * 256
        def hist_body(c, carry):
            k16 = keys[pl.ds(c * 16, 16)]
            slot = lane256 + k16
            cnt = plsc.load_gather(hist2d, [slot])
            plsc.store_scatter(hist2d, [slot], cnt + ones16)
            return carry
        lax.fori_loop(0, S // 16, hist_body, 0, unroll=4)

        carry_in = jnp.int32(0)
        for g in range(16):
            tot = zero16
            for l in range(16):
                tot = tot + hist2d[pl.ds(l * 256 + g * 16, 16)]
            incl = plsc.cumsum(tot)
            excl = incl - tot + carry_in
            for l in range(16):
                offs[g * 16 + l] = excl[l]
            carry_in = carry_in + incl[15]

        def place_body(c, carry):
            k16 = keys[pl.ds(c * 16, 16)]
            src16 = src_base + c * 16 + lane
            rvec = zero16
            for l in range(16):
                k = k16[l]
                r = offs[k]
                offs[k] = r + 1
                rvec = jnp.where(lane == l, r, rvec)
            plsc.store_scatter(gidx, [rvec], src16)
            return carry
        lax.fori_loop(0, S // 16, place_body, 0)

        for c in range(N_CHUNKS):
            for g in range(CHUNK // 16):
                t0 = c * CHUNK + g * 16
                sidx[c, pl.ds(g * 16, 16)] = (
                    dst_base + (t0 + lane) * NUM_HEADS)

        qbufs = (qr0, qr1)
        kvbufs = (kvr0, kvr1)
        gqs = (gq0, gq1)
        gkvs = (gkv0, gkv1)
        sqs = (sq0, sq1)
        skvs = (skv0, skv1)

        def chunk_step(c, p):
            @pl.when(c >= 2)
            def _():
                pltpu.make_async_copy(qbufs[p], qs_hbm.at[sidx.at[c - 2]],
                                      sqs[p]).wait()
                pltpu.make_async_copy(kvbufs[p], kvs_hbm.at[sidx.at[c - 2]],
                                      skvs[p]).wait()

            gi = gidx.at[pl.ds(c * CHUNK, CHUNK)]
            pltpu.make_async_copy(q_hbm.at[gi], qbufs[p], gqs[p]).start()
            pltpu.make_async_copy(kv_hbm.at[gi], kvbufs[p], gkvs[p]).start()
            pltpu.make_async_copy(q_hbm.at[gi], qbufs[p], gqs[p]).wait()
            pltpu.make_async_copy(kv_hbm.at[gi], kvbufs[p], gkvs[p]).wait()
            pltpu.make_async_copy(qbufs[p], qs_hbm.at[sidx.at[c]],
                                  sqs[p]).start()
            pltpu.make_async_copy(kvbufs[p], kvs_hbm.at[sidx.at[c]],
                                  skvs[p]).start()

        def outer(c, carry):
            chunk_step(c * 2, 0)
            chunk_step(c * 2 + 1, 1)
            return carry
        lax.fori_loop(0, N_CHUNKS // 2, outer, 0)

        for p in range(2):
            c = N_CHUNKS - 2 + p
            pltpu.make_async_copy(qbufs[p], qs_hbm.at[sidx.at[c]],
                                  sqs[p]).wait()
            pltpu.make_async_copy(kvbufs[p], kvs_hbm.at[sidx.at[c]],
                                  skvs[p]).wait()

    return sc_kernel(buckets_flat, q_table, kv_table)


def _attn_body(q_ref, kv_ref, wo_ref, bo_ref, out_ref, att_ref):
    n_groups = q_ref.shape[0]

    r = lax.broadcasted_iota(jnp.int32, (GROUP * NUM_HEADS,
                                         GROUP * NUM_HEADS), 0)
    c = lax.broadcasted_iota(jnp.int32, (GROUP * NUM_HEADS,
                                         GROUP * NUM_HEADS), 1)
    same_pos = (r // NUM_HEADS) == (c // NUM_HEADS)

    def unpk_lo(p):
        return pltpu.bitcast(lax.shift_left(p, 16),
                             jnp.float32).astype(jnp.bfloat16)

    def unpk_hi(p):
        return pltpu.bitcast(jnp.bitwise_and(p, MASK_HI),
                             jnp.float32).astype(jnp.bfloat16)

    for g in range(n_groups):
        qg = q_ref[g].astype(jnp.bfloat16)
        kvp = kv_ref[g]
        kg = unpk_lo(kvp)
        vg = unpk_hi(kvp)
        s = lax.dot_general(
            qg, kg, (((1,), (1,)), ((), ())),
            preferred_element_type=jnp.float32) * SCALE
        s = jnp.where(same_pos, s, -jnp.inf)
        s = s - jnp.max(s, axis=-1, keepdims=True)
        e = jnp.exp(s)
        p = (e / jnp.sum(e, axis=-1, keepdims=True)).astype(jnp.bfloat16)
        og = jnp.dot(p, vg, preferred_element_type=jnp.float32)
        att_ref[g * GROUP:(g + 1) * GROUP, :] = og.astype(
            jnp.bfloat16).reshape(GROUP, HIDDEN)

    out_ref[...] = (
        jnp.dot(att_ref[...], wo_ref[...], preferred_element_type=jnp.float32)
        + bo_ref[...])


def _attn_proj(q_s, kv_s, wo_bf16, bo, bm):
    m_tiles = (B * S) // bm
    g = bm // GROUP
    return pl.pallas_call(
        _attn_body,
        grid=(m_tiles,),
        in_specs=[
            pl.BlockSpec((g, GROUP * NUM_HEADS, HEAD_DIM),
                         lambda m: (m, 0, 0)),
            pl.BlockSpec((g, GROUP * NUM_HEADS, KVW), lambda m: (m, 0, 0)),
            pl.BlockSpec((HIDDEN, HIDDEN), lambda m: (0, 0)),
            pl.BlockSpec((1, HIDDEN), lambda m: (0, 0)),
        ],
        out_specs=pl.BlockSpec((bm, HIDDEN), lambda m: (m, 0)),
        out_shape=jax.ShapeDtypeStruct((B * S, HIDDEN), jnp.float32),
        scratch_shapes=[pltpu.VMEM((bm, HIDDEN), jnp.bfloat16)],
    )(q_s, kv_s, wo_bf16, bo.reshape(1, -1))


@jax.jit
def kernel(x, Wq, bq, Wk, bk, Wv, bv, Wo, bo, proj):
    x2d = x.reshape(B * S, HIDDEN)
    xb = x2d.astype(jnp.bfloat16)
    wkv = jnp.concatenate(
        [Wk.reshape(HIDDEN, NUM_HEADS, 1, HEAD_DIM),
         Wv.reshape(HIDDEN, NUM_HEADS, 1, HEAD_DIM)], axis=2,
    ).reshape(HIDDEN, 2 * HIDDEN).astype(jnp.bfloat16)
    bkv = jnp.concatenate(
        [bk.reshape(NUM_HEADS, 1, HEAD_DIM), bv.reshape(NUM_HEADS, 1,
                                                        HEAD_DIM)], axis=1,
    ).reshape(2 * HIDDEN)

    q_tab, bk6 = _q_proj(x2d, Wq, bq, proj, bm=512, bn=1024)
    kv_tab = _kv_proj(xb, wkv, bkv, bm=512, bn=2048)

    buckets = bk6.reshape(NUM_HEADS, B, S).swapaxes(0, 1).reshape(-1)

    q_s, kv_s = _sc_sort_gather(
        buckets,
        q_tab.reshape(B * NUM_HEADS * S, HEAD_DIM),
        kv_tab.reshape(B * NUM_HEADS * S, KVW))
    q_s = q_s.reshape(B * S // GROUP, GROUP * NUM_HEADS, HEAD_DIM)
    kv_s = kv_s.reshape(B * S // GROUP, GROUP * NUM_HEADS, KVW)

    out = _attn_proj(q_s, kv_s, Wo.astype(jnp.bfloat16), bo, bm=256)
    return out.reshape(B, S, HIDDEN)


# speedup vs baseline: 1.1948x; 1.1251x over previous
import functools

import jax
import jax.numpy as jnp
from jax import lax
from jax.experimental import pallas as pl
from jax.experimental.pallas import tpu as pltpu
from jax.experimental.pallas import tpu_sc as plsc

NUM_HEADS = 16
HEAD_DIM = 128
HIDDEN = 2048
NUM_HASHES = 8
SCALE = HEAD_DIM ** (-0.5)
B, S = 2, 4096
GROUP = 8
KV = 2 * HEAD_DIM
KVW = HEAD_DIM
CHUNK = 128
N_CHUNKS = S // CHUNK


def _q_body(x_ref, w_ref, b_ref, projt_ref, q_ref, bk_ref):
    acc = jnp.dot(x_ref[...], w_ref[...], preferred_element_type=jnp.float32)
    acc = acc + b_ref[...]
    bm = acc.shape[0]
    heads = acc.shape[1] // HEAD_DIM
    q_ref[...] = acc.reshape(bm, heads, HEAD_DIM).swapaxes(0, 1)[None]
    powers = (2 ** lax.iota(jnp.int32, NUM_HASHES))[None, :]
    for hh in range(heads):
        qp = jnp.dot(acc[:, hh * HEAD_DIM:(hh + 1) * HEAD_DIM],
                     projt_ref[...], preferred_element_type=jnp.float32)
        bits = (qp > 0).astype(jnp.int32)
        bk_ref[0, hh, :] = jnp.sum(bits * powers, axis=-1)


def _q_proj(x2d, wq, bq, proj, bm, bn):
    m_tiles = x2d.shape[0] // bm
    n_tiles = HIDDEN // bn
    heads_per_n = bn // HEAD_DIM
    s_tiles = S // bm
    return pl.pallas_call(
        _q_body,
        grid=(n_tiles, m_tiles),
        in_specs=[
            pl.BlockSpec((bm, HIDDEN), lambda n, m: (m, 0)),
            pl.BlockSpec((HIDDEN, bn), lambda n, m: (0, n)),
            pl.BlockSpec((1, bn), lambda n, m: (0, n)),
            pl.BlockSpec((HEAD_DIM, NUM_HASHES), lambda n, m: (0, 0)),
        ],
        out_specs=[
            pl.BlockSpec((1, heads_per_n, bm, HEAD_DIM),
                         lambda n, m: (m // s_tiles, n, m % s_tiles, 0)),
            pl.BlockSpec((1, heads_per_n, bm), lambda n, m: (n, 0, m)),
        ],
        out_shape=[
            jax.ShapeDtypeStruct((B, NUM_HEADS, S, HEAD_DIM), jnp.float32),
            jax.ShapeDtypeStruct((n_tiles, heads_per_n, B * S), jnp.int32),
        ],
    )(x2d, wq, bq.reshape(1, -1), proj.T)


def _kv_body(x_ref, w_ref, b_ref, out_ref):
    acc = jnp.dot(x_ref[...], w_ref[...], preferred_element_type=jnp.float32)
    acc = acc + b_ref[...]
    bm = acc.shape[0]
    heads = acc.shape[1] // KV
    for hh in range(heads):
        kv = acc[:, hh * KV:(hh + 1) * KV].reshape(bm, 2, HEAD_DIM)
        inter = kv.reshape(2 * bm, HEAD_DIM).astype(jnp.bfloat16)
        out_ref[0, hh] = pltpu.bitcast(inter, jnp.int32)


def _kv_proj(xb, wkv, bkv, bm, bn):
    m_tiles = xb.shape[0] // bm
    n_tiles = wkv.shape[1] // bn
    heads_per_n = bn // KV
    s_tiles = S // bm
    return pl.pallas_call(
        _kv_body,
        grid=(n_tiles, m_tiles),
        in_specs=[
            pl.BlockSpec((bm, HIDDEN), lambda n, m: (m, 0)),
            pl.BlockSpec((HIDDEN, bn), lambda n, m: (0, n)),
            pl.BlockSpec((1, bn), lambda n, m: (0, n)),
        ],
        out_specs=pl.BlockSpec(
            (1, heads_per_n, bm, KVW),
            lambda n, m: (m // s_tiles, n, m % s_tiles, 0)),
        out_shape=jax.ShapeDtypeStruct((B, NUM_HEADS, S, KVW), jnp.int32),
    )(xb, wkv, bkv.reshape(1, -1))


def _sc_sort_gather(buckets_flat, q_table, kv_table):
    mesh = plsc.VectorSubcoreMesh(core_axis_name="c", subcore_axis_name="s")

    @functools.partial(
        pl.kernel,
        out_type=[
            jax.ShapeDtypeStruct((B * S * NUM_HEADS, HEAD_DIM), jnp.float32),
            jax.ShapeDtypeStruct((B * S * NUM_HEADS, KVW), jnp.int32),
        ],
        mesh=mesh,
        scratch_types=[
            pltpu.VMEM((S,), jnp.int32),
            pltpu.VMEM((16 * 256,), jnp.int32),
            pltpu.VMEM((S,), jnp.int32),
            pltpu.VMEM((N_CHUNKS, CHUNK), jnp.int32),
            pltpu.VMEM((CHUNK, HEAD_DIM), jnp.float32),
            pltpu.VMEM((CHUNK, HEAD_DIM), jnp.float32),
            pltpu.VMEM((CHUNK, KVW), jnp.int32),
            pltpu.VMEM((CHUNK, KVW), jnp.int32),
            pltpu.SMEM((256,), jnp.int32),
            pltpu.SemaphoreType.DMA,
            pltpu.SemaphoreType.DMA,
            pltpu.SemaphoreType.DMA,
            pltpu.SemaphoreType.DMA,
            pltpu.SemaphoreType.DMA,
            pltpu.SemaphoreType.DMA,
            pltpu.SemaphoreType.DMA,
            pltpu.SemaphoreType.DMA,
        ],
        compiler_params=pltpu.CompilerParams(needs_layout_passes=False),
    )
    def sc_kernel(buckets_hbm, q_hbm, kv_hbm, qs_hbm, kvs_hbm,
                  keys, hist2d, gidx, sidx, qr0, qr1, kvr0, kvr1, offs,
                  gq0, gq1, gkv0, gkv1, sq0, sq1, skv0, skv1):
        w = lax.axis_index("s") * 2 + lax.axis_index("c")
        b = w // NUM_HEADS
        h = w % NUM_HEADS
        src_base = w * S
        dst_base = b * S * NUM_HEADS + h

        lane = lax.iota(jnp.int32, 16)
        zero16 = jnp.zeros((16,), jnp.int32)
        ones16 = jnp.ones((16,), jnp.int32)

        pltpu.sync_copy(buckets_hbm.at[pl.ds(w * S, S)], keys)

        for j in range(16 * 256 // 16):
            hist2d[pl.ds(j * 16, 16)] = zero16

        lane256 = lane * 256
        def hist_body(c, carry):
            k16 = keys[pl.ds(c * 16, 16)]
            slot = lane256 + k16
            cnt = plsc.load_gather(hist2d, [slot])
            plsc.store_scatter(hist2d, [slot], cnt + ones16)
            return carry
        lax.fori_loop(0, S // 16, hist_body, 0, unroll=4)

        carry_in = jnp.int32(0)
        for g in range(16):
            tot = zero16
            for l in range(16):
                tot = tot + hist2d[pl.ds(l * 256 + g * 16, 16)]
            incl = plsc.cumsum(tot)
            excl = incl - tot + carry_in
            for l in range(16):
                offs[g * 16 + l] = excl[l]
            carry_in = carry_in + incl[15]

        def place_body(c, carry):
            k16 = keys[pl.ds(c * 16, 16)]
            src16 = src_base + c * 16 + lane
            rvec = zero16
            for l in range(16):
                k = k16[l]
                r = offs[k]
                offs[k] = r + 1
                rvec = jnp.where(lane == l, r, rvec)
            plsc.store_scatter(gidx, [rvec], src16)
            return carry
        lax.fori_loop(0, S // 16, place_body, 0)

        for c in range(N_CHUNKS):
            for g in range(CHUNK // 16):
                t0 = c * CHUNK + g * 16
                sidx[c, pl.ds(g * 16, 16)] = (
                    dst_base + (t0 + lane) * NUM_HEADS)

        qbufs = (qr0, qr1)
        kvbufs = (kvr0, kvr1)
        gqs = (gq0, gq1)
        gkvs = (gkv0, gkv1)
        sqs = (sq0, sq1)
        skvs = (skv0, skv1)

        def chunk_step(c, p):
            @pl.when(c >= 2)
            def _():
                pltpu.make_async_copy(qbufs[p], qs_hbm.at[sidx.at[c - 2]],
                                      sqs[p]).wait()
                pltpu.make_async_copy(kvbufs[p], kvs_hbm.at[sidx.at[c - 2]],
                                      skvs[p]).wait()

            gi = gidx.at[pl.ds(c * CHUNK, CHUNK)]
            pltpu.make_async_copy(q_hbm.at[gi], qbufs[p], gqs[p]).start()
            pltpu.make_async_copy(kv_hbm.at[gi], kvbufs[p], gkvs[p]).start()
            pltpu.make_async_copy(q_hbm.at[gi], qbufs[p], gqs[p]).wait()
            pltpu.make_async_copy(kv_hbm.at[gi], kvbufs[p], gkvs[p]).wait()
            pltpu.make_async_copy(qbufs[p], qs_hbm.at[sidx.at[c]],
                                  sqs[p]).start()
            pltpu.make_async_copy(kvbufs[p], kvs_hbm.at[sidx.at[c]],
                                  skvs[p]).start()

        def outer(c, carry):
            chunk_step(c * 2, 0)
            chunk_step(c * 2 + 1, 1)
            return carry
        lax.fori_loop(0, N_CHUNKS // 2, outer, 0)

        for p in range(2):
            c = N_CHUNKS - 2 + p
            pltpu.make_async_copy(qbufs[p], qs_hbm.at[sidx.at[c]],
                                  sqs[p]).wait()
            pltpu.make_async_copy(kvbufs[p], kvs_hbm.at[sidx.at[c]],
                                  skvs[p]).wait()

    return sc_kernel(buckets_flat, q_table, kv_table)


def _attn_body(q_ref, kv_ref, wo_ref, bo_ref, out_ref, att_ref,
               s_ref, p_ref, k_ref, v_ref):
    n_groups = q_ref.shape[0]
    R = GROUP * NUM_HEADS

    kvb = pltpu.bitcast(kv_ref[...], jnp.bfloat16).reshape(
        n_groups, R, 2, HEAD_DIM)
    k_ref[...] = kvb[:, :, 0, :].reshape(n_groups * R, HEAD_DIM)
    v_ref[...] = kvb[:, :, 1, :].reshape(n_groups * R, HEAD_DIM)

    for g in range(n_groups):
        qg = q_ref[g].astype(jnp.bfloat16)
        s_ref[g * R:(g + 1) * R, :] = lax.dot_general(
            qg, k_ref[g * R:(g + 1) * R, :], (((1,), (1,)), ((), ())),
            preferred_element_type=jnp.float32)

    rows = n_groups * R
    r = lax.broadcasted_iota(jnp.int32, (rows, R), 0)
    c = lax.broadcasted_iota(jnp.int32, (rows, R), 1)
    same_pos = ((r % R) // NUM_HEADS) == (c // NUM_HEADS)
    s = jnp.where(same_pos, s_ref[...] * SCALE, -jnp.inf)
    s = s - jnp.max(s, axis=-1, keepdims=True)
    e = jnp.exp(s)
    p_ref[...] = (e / jnp.sum(e, axis=-1, keepdims=True)).astype(jnp.bfloat16)

    for g in range(n_groups):
        og = jnp.dot(p_ref[g * R:(g + 1) * R, :],
                     v_ref[g * R:(g + 1) * R, :],
                     preferred_element_type=jnp.float32)
        att_ref[g * GROUP:(g + 1) * GROUP, :] = og.astype(
            jnp.bfloat16).reshape(GROUP, HIDDEN)

    out_ref[...] = (
        jnp.dot(att_ref[...], wo_ref[...], preferred_element_type=jnp.float32)
        + bo_ref[...])


def _attn_proj(q_s, kv_s, wo_bf16, bo, bm):
    m_tiles = (B * S) // bm
    g = bm // GROUP
    return pl.pallas_call(
        _attn_body,
        grid=(m_tiles,),
        in_specs=[
            pl.BlockSpec((g, GROUP * NUM_HEADS, HEAD_DIM),
                         lambda m: (m, 0, 0)),
            pl.BlockSpec((g, GROUP * NUM_HEADS, KVW), lambda m: (m, 0, 0)),
            pl.BlockSpec((HIDDEN, HIDDEN), lambda m: (0, 0)),
            pl.BlockSpec((1, HIDDEN), lambda m: (0, 0)),
        ],
        out_specs=pl.BlockSpec((bm, HIDDEN), lambda m: (m, 0)),
        out_shape=jax.ShapeDtypeStruct((B * S, HIDDEN), jnp.float32),
        scratch_shapes=[
            pltpu.VMEM((bm, HIDDEN), jnp.bfloat16),
            pltpu.VMEM((g * GROUP * NUM_HEADS, GROUP * NUM_HEADS),
                       jnp.float32),
            pltpu.VMEM((g * GROUP * NUM_HEADS, GROUP * NUM_HEADS),
                       jnp.bfloat16),
            pltpu.VMEM((g * GROUP * NUM_HEADS, HEAD_DIM), jnp.bfloat16),
            pltpu.VMEM((g * GROUP * NUM_HEADS, HEAD_DIM), jnp.bfloat16),
        ],
    )(q_s, kv_s, wo_bf16, bo.reshape(1, -1))


@jax.jit
def kernel(x, Wq, bq, Wk, bk, Wv, bv, Wo, bo, proj):
    x2d = x.reshape(B * S, HIDDEN)
    xb = x2d.astype(jnp.bfloat16)
    wkv = jnp.concatenate(
        [Wk.reshape(HIDDEN, NUM_HEADS, 1, HEAD_DIM),
         Wv.reshape(HIDDEN, NUM_HEADS, 1, HEAD_DIM)], axis=2,
    ).reshape(HIDDEN, 2 * HIDDEN).astype(jnp.bfloat16)
    bkv = jnp.concatenate(
        [bk.reshape(NUM_HEADS, 1, HEAD_DIM), bv.reshape(NUM_HEADS, 1,
                                                        HEAD_DIM)], axis=1,
    ).reshape(2 * HIDDEN)

    q_tab, bk6 = _q_proj(x2d, Wq, bq, proj, bm=512, bn=1024)
    kv_tab = _kv_proj(xb, wkv, bkv, bm=512, bn=2048)

    buckets = bk6.reshape(NUM_HEADS, B, S).swapaxes(0, 1).reshape(-1)

    q_s, kv_s = _sc_sort_gather(
        buckets,
        q_tab.reshape(B * NUM_HEADS * S, HEAD_DIM),
        kv_tab.reshape(B * NUM_HEADS * S, KVW))
    q_s = q_s.reshape(B * S // GROUP, GROUP * NUM_HEADS, HEAD_DIM)
    kv_s = kv_s.reshape(B * S // GROUP, GROUP * NUM_HEADS, KVW)

    out = _attn_proj(q_s, kv_s, Wo.astype(jnp.bfloat16), bo, bm=256)
    return out.reshape(B, S, HIDDEN)


# speedup vs baseline: 1.3111x; 1.0973x over previous
import functools

import jax
import jax.numpy as jnp
from jax import lax
from jax.experimental import pallas as pl
from jax.experimental.pallas import tpu as pltpu
from jax.experimental.pallas import tpu_sc as plsc

NUM_HEADS = 16
HEAD_DIM = 128
HIDDEN = 2048
NUM_HASHES = 8
SCALE = HEAD_DIM ** (-0.5)
B, S = 2, 4096
GROUP = 8
KV = 2 * HEAD_DIM
KVW = HEAD_DIM
CHUNK = 128
N_CHUNKS = S // CHUNK


def _q_body(x_ref, w_ref, b_ref, projbd_ref, powbd_ref, q_ref, bk_ref):
    acc = jnp.dot(x_ref[...], w_ref[...], preferred_element_type=jnp.float32)
    acc = acc + b_ref[...]
    bm = acc.shape[0]
    heads = acc.shape[1] // HEAD_DIM
    q_ref[...] = acc.reshape(bm, heads, HEAD_DIM)
    qp = jnp.dot(acc, projbd_ref[...], preferred_element_type=jnp.float32)
    bits = (qp > 0).astype(jnp.bfloat16)
    bk = jnp.dot(bits, powbd_ref[...], preferred_element_type=jnp.float32)
    bk_ref[...] = bk.astype(jnp.int32)[None]


def _q_proj(x2d, wq, bq, projbd, powbd, bm, bn):
    m_tiles = x2d.shape[0] // bm
    n_tiles = HIDDEN // bn
    heads_per_n = bn // HEAD_DIM
    return pl.pallas_call(
        _q_body,
        grid=(n_tiles, m_tiles),
        in_specs=[
            pl.BlockSpec((bm, HIDDEN), lambda n, m: (m, 0)),
            pl.BlockSpec((HIDDEN, bn), lambda n, m: (0, n)),
            pl.BlockSpec((1, bn), lambda n, m: (0, n)),
            pl.BlockSpec((bn, NUM_HASHES * heads_per_n), lambda n, m: (0, 0)),
            pl.BlockSpec((NUM_HASHES * heads_per_n, heads_per_n),
                         lambda n, m: (0, 0)),
        ],
        out_specs=[
            pl.BlockSpec((bm, heads_per_n, HEAD_DIM), lambda n, m: (m, n, 0)),
            pl.BlockSpec((1, bm, heads_per_n), lambda n, m: (n, m, 0)),
        ],
        out_shape=[
            jax.ShapeDtypeStruct((B * S, NUM_HEADS, HEAD_DIM), jnp.float32),
            jax.ShapeDtypeStruct((n_tiles, B * S, heads_per_n), jnp.int32),
        ],
    )(x2d, wq, bq.reshape(1, -1), projbd, powbd)


def _kv_body(x_ref, w_ref, b_ref, out_ref):
    acc = jnp.dot(x_ref[...], w_ref[...], preferred_element_type=jnp.float32)
    acc = acc + b_ref[...]
    bm = acc.shape[0]
    heads = acc.shape[1] // KV
    for hh in range(heads):
        kv = acc[:, hh * KV:(hh + 1) * KV].reshape(bm, 2, HEAD_DIM)
        inter = kv.reshape(2 * bm, HEAD_DIM).astype(jnp.bfloat16)
        out_ref[:, hh] = pltpu.bitcast(inter, jnp.int32)


def _kv_proj(xb, wkv, bkv, bm, bn):
    m_tiles = xb.shape[0] // bm
    n_tiles = wkv.shape[1] // bn
    heads_per_n = bn // KV
    return pl.pallas_call(
        _kv_body,
        grid=(n_tiles, m_tiles),
        in_specs=[
            pl.BlockSpec((bm, HIDDEN), lambda n, m: (m, 0)),
            pl.BlockSpec((HIDDEN, bn), lambda n, m: (0, n)),
            pl.BlockSpec((1, bn), lambda n, m: (0, n)),
        ],
        out_specs=pl.BlockSpec(
            (bm, heads_per_n, KVW), lambda n, m: (m, n, 0)),
        out_shape=jax.ShapeDtypeStruct((B * S, NUM_HEADS, KVW), jnp.int32),
    )(xb, wkv, bkv.reshape(1, -1))


def _sc_sort_gather(buckets_flat, q_table, kv_table):
    mesh = plsc.VectorSubcoreMesh(core_axis_name="c", subcore_axis_name="s")

    @functools.partial(
        pl.kernel,
        out_type=[
            jax.ShapeDtypeStruct((B * S * NUM_HEADS, HEAD_DIM), jnp.float32),
            jax.ShapeDtypeStruct((B * S * NUM_HEADS, KVW), jnp.int32),
        ],
        mesh=mesh,
        scratch_types=[
            pltpu.VMEM((S,), jnp.int32),
            pltpu.VMEM((16 * 256,), jnp.int32),
            pltpu.VMEM((S,), jnp.int32),
            pltpu.VMEM((N_CHUNKS, CHUNK), jnp.int32),
            pltpu.VMEM((CHUNK, HEAD_DIM), jnp.float32),
            pltpu.VMEM((CHUNK, HEAD_DIM), jnp.float32),
            pltpu.VMEM((CHUNK, KVW), jnp.int32),
            pltpu.VMEM((CHUNK, KVW), jnp.int32),
            pltpu.SMEM((256,), jnp.int32),
            pltpu.SemaphoreType.DMA,
            pltpu.SemaphoreType.DMA,
            pltpu.SemaphoreType.DMA,
            pltpu.SemaphoreType.DMA,
            pltpu.SemaphoreType.DMA,
            pltpu.SemaphoreType.DMA,
            pltpu.SemaphoreType.DMA,
            pltpu.SemaphoreType.DMA,
        ],
        compiler_params=pltpu.CompilerParams(needs_layout_passes=False),
    )
    def sc_kernel(buckets_hbm, q_hbm, kv_hbm, qs_hbm, kvs_hbm,
                  keys, hist2d, gidx, sidx, qr0, qr1, kvr0, kvr1, offs,
                  gq0, gq1, gkv0, gkv1, sq0, sq1, skv0, skv1):
        w = lax.axis_index("s") * 2 + lax.axis_index("c")
        b = w // NUM_HEADS
        h = w % NUM_HEADS
        base = b * S * NUM_HEADS + h

        lane = lax.iota(jnp.int32, 16)
        zero16 = jnp.zeros((16,), jnp.int32)
        ones16 = jnp.ones((16,), jnp.int32)

        pltpu.sync_copy(buckets_hbm.at[pl.ds(w * S, S)], keys)

        for j in range(16 * 256 // 16):
            hist2d[pl.ds(j * 16, 16)] = zero16

        lane256 = lane * 256
        def hist_body(c, carry):
            k16 = keys[pl.ds(c * 16, 16)]
            slot = lane256 + k16
            cnt = plsc.load_gather(hist2d, [slot])
            plsc.store_scatter(hist2d, [slot], cnt + ones16)
            return carry
        lax.fori_loop(0, S // 16, hist_body, 0, unroll=4)

        carry_in = jnp.int32(0)
        for g in range(16):
            tot = zero16
            for l in range(16):
                tot = tot + hist2d[pl.ds(l * 256 + g * 16, 16)]
            incl = plsc.cumsum(tot)
            excl = incl - tot + carry_in
            for l in range(16):
                offs[g * 16 + l] = excl[l]
            carry_in = carry_in + incl[15]

        def place_body(c, carry):
            k16 = keys[pl.ds(c * 16, 16)]
            src16 = base + (c * 16 + lane) * NUM_HEADS
            rvec = zero16
            for l in range(16):
                k = k16[l]
                r = offs[k]
                offs[k] = r + 1
                rvec = jnp.where(lane == l, r, rvec)
            plsc.store_scatter(gidx, [rvec], src16)
            return carry
        lax.fori_loop(0, S // 16, place_body, 0)

        for c in range(N_CHUNKS):
            for g in range(CHUNK // 16):
                t0 = c * CHUNK + g * 16
                sidx[c, pl.ds(g * 16, 16)] = (
                    base + (t0 + lane) * NUM_HEADS)

        qbufs = (qr0, qr1)
        kvbufs = (kvr0, kvr1)
        gqs = (gq0, gq1)
        gkvs = (gkv0, gkv1)
        sqs = (sq0, sq1)
        skvs = (skv0, skv1)

        def chunk_step(c, p):
            @pl.when(c >= 2)
            def _():
                pltpu.make_async_copy(qbufs[p], qs_hbm.at[sidx.at[c - 2]],
                                      sqs[p]).wait()
                pltpu.make_async_copy(kvbufs[p], kvs_hbm.at[sidx.at[c - 2]],
                                      skvs[p]).wait()

            gi = gidx.at[pl.ds(c * CHUNK, CHUNK)]
            pltpu.make_async_copy(q_hbm.at[gi], qbufs[p], gqs[p]).start()
            pltpu.make_async_copy(kv_hbm.at[gi], kvbufs[p], gkvs[p]).start()
            pltpu.make_async_copy(q_hbm.at[gi], qbufs[p], gqs[p]).wait()
            pltpu.make_async_copy(kv_hbm.at[gi], kvbufs[p], gkvs[p]).wait()
            pltpu.make_async_copy(qbufs[p], qs_hbm.at[sidx.at[c]],
                                  sqs[p]).start()
            pltpu.make_async_copy(kvbufs[p], kvs_hbm.at[sidx.at[c]],
                                  skvs[p]).start()

        def outer(c, carry):
            chunk_step(c * 2, 0)
            chunk_step(c * 2 + 1, 1)
            return carry
        lax.fori_loop(0, N_CHUNKS // 2, outer, 0)

        for p in range(2):
            c = N_CHUNKS - 2 + p
            pltpu.make_async_copy(qbufs[p], qs_hbm.at[sidx.at[c]],
                                  sqs[p]).wait()
            pltpu.make_async_copy(kvbufs[p], kvs_hbm.at[sidx.at[c]],
                                  skvs[p]).wait()

    return sc_kernel(buckets_flat, q_table, kv_table)


def _attn_body(q_ref, kv_ref, wo_ref, bo_ref, out_ref, att_ref,
               s_ref, p_ref, k_ref, v_ref):
    n_groups = q_ref.shape[0]
    R = GROUP * NUM_HEADS

    kvb = pltpu.bitcast(kv_ref[...], jnp.bfloat16).reshape(
        n_groups, R, 2, HEAD_DIM)
    k_ref[...] = kvb[:, :, 0, :].reshape(n_groups * R, HEAD_DIM)
    v_ref[...] = kvb[:, :, 1, :].reshape(n_groups * R, HEAD_DIM)

    for g in range(n_groups):
        qg = q_ref[g].astype(jnp.bfloat16)
        s_ref[g * R:(g + 1) * R, :] = lax.dot_general(
            qg, k_ref[g * R:(g + 1) * R, :], (((1,), (1,)), ((), ())),
            preferred_element_type=jnp.float32)

    rows = n_groups * R
    r = lax.broadcasted_iota(jnp.int32, (rows, R), 0)
    c = lax.broadcasted_iota(jnp.int32, (rows, R), 1)
    same_pos = ((r % R) // NUM_HEADS) == (c // NUM_HEADS)
    s = jnp.where(same_pos, s_ref[...] * SCALE, -jnp.inf)
    s = s - jnp.max(s, axis=-1, keepdims=True)
    e = jnp.exp(s)
    p_ref[...] = (e / jnp.sum(e, axis=-1, keepdims=True)).astype(jnp.bfloat16)

    for g in range(n_groups):
        og = jnp.dot(p_ref[g * R:(g + 1) * R, :],
                     v_ref[g * R:(g + 1) * R, :],
                     preferred_element_type=jnp.float32)
        att_ref[g * GROUP:(g + 1) * GROUP, :] = og.astype(
            jnp.bfloat16).reshape(GROUP, HIDDEN)

    out_ref[...] = (
        jnp.dot(att_ref[...], wo_ref[...], preferred_element_type=jnp.float32)
        + bo_ref[...])


def _attn_proj(q_s, kv_s, wo_bf16, bo, bm):
    m_tiles = (B * S) // bm
    g = bm // GROUP
    return pl.pallas_call(
        _attn_body,
        grid=(m_tiles,),
        in_specs=[
            pl.BlockSpec((g, GROUP * NUM_HEADS, HEAD_DIM),
                         lambda m: (m, 0, 0)),
            pl.BlockSpec((g, GROUP * NUM_HEADS, KVW), lambda m: (m, 0, 0)),
            pl.BlockSpec((HIDDEN, HIDDEN), lambda m: (0, 0)),
            pl.BlockSpec((1, HIDDEN), lambda m: (0, 0)),
        ],
        out_specs=pl.BlockSpec((bm, HIDDEN), lambda m: (m, 0)),
        out_shape=jax.ShapeDtypeStruct((B * S, HIDDEN), jnp.float32),
        scratch_shapes=[
            pltpu.VMEM((bm, HIDDEN), jnp.bfloat16),
            pltpu.VMEM((g * GROUP * NUM_HEADS, GROUP * NUM_HEADS),
                       jnp.float32),
            pltpu.VMEM((g * GROUP * NUM_HEADS, GROUP * NUM_HEADS),
                       jnp.bfloat16),
            pltpu.VMEM((g * GROUP * NUM_HEADS, HEAD_DIM), jnp.bfloat16),
            pltpu.VMEM((g * GROUP * NUM_HEADS, HEAD_DIM), jnp.bfloat16),
        ],
    )(q_s, kv_s, wo_bf16, bo.reshape(1, -1))


@jax.jit
def kernel(x, Wq, bq, Wk, bk, Wv, bv, Wo, bo, proj):
    x2d = x.reshape(B * S, HIDDEN)
    xb = x2d.astype(jnp.bfloat16)
    wkv = jnp.concatenate(
        [Wk.reshape(HIDDEN, NUM_HEADS, 1, HEAD_DIM),
         Wv.reshape(HIDDEN, NUM_HEADS, 1, HEAD_DIM)], axis=2,
    ).reshape(HIDDEN, 2 * HIDDEN).astype(jnp.bfloat16)
    bkv = jnp.concatenate(
        [bk.reshape(NUM_HEADS, 1, HEAD_DIM), bv.reshape(NUM_HEADS, 1,
                                                        HEAD_DIM)], axis=1,
    ).reshape(2 * HIDDEN)

    heads_per_n = 1024 // HEAD_DIM
    projbd = jnp.kron(jnp.eye(heads_per_n, dtype=jnp.float32), proj.T)
    powers = (2.0 ** jnp.arange(NUM_HASHES, dtype=jnp.float32))[:, None]
    powbd = jnp.kron(jnp.eye(heads_per_n, dtype=jnp.float32),
                     powers).astype(jnp.bfloat16)

    q_tab, bk = _q_proj(x2d, Wq, bq, projbd, powbd, bm=512, bn=1024)
    kv_tab = _kv_proj(xb, wkv, bkv, bm=512, bn=2048)

    buckets = bk.reshape(2, B, S, NUM_HEADS // 2).transpose(
        (1, 0, 3, 2)).reshape(-1)

    q_s, kv_s = _sc_sort_gather(
        buckets,
        q_tab.reshape(B * NUM_HEADS * S, HEAD_DIM),
        kv_tab.reshape(B * NUM_HEADS * S, KVW))
    q_s = q_s.reshape(B * S // GROUP, GROUP * NUM_HEADS, HEAD_DIM)
    kv_s = kv_s.reshape(B * S // GROUP, GROUP * NUM_HEADS, KVW)

    out = _attn_proj(q_s, kv_s, Wo.astype(jnp.bfloat16), bo, bm=256)
    return out.reshape(B, S, HIDDEN)


# speedup vs baseline: 1.4099x; 1.0754x over previous
import functools

import jax
import jax.numpy as jnp
from jax import lax
from jax.experimental import pallas as pl
from jax.experimental.pallas import tpu as pltpu
from jax.experimental.pallas import tpu_sc as plsc

NUM_HEADS = 16
HEAD_DIM = 128
HIDDEN = 2048
NUM_HASHES = 8
SCALE = HEAD_DIM ** (-0.5)
B, S = 2, 4096
GROUP = 8
KV = 2 * HEAD_DIM
KVW = HEAD_DIM
CHUNK = 128
N_CHUNKS = S // CHUNK


def _q_body(x_ref, w_ref, b_ref, projbd_ref, powbd_ref, q_ref, bk_ref):
    acc = jnp.dot(x_ref[...], w_ref[...], preferred_element_type=jnp.float32)
    acc = acc + b_ref[...]
    bm = acc.shape[0]
    heads = acc.shape[1] // HEAD_DIM
    q_ref[...] = acc.reshape(bm, heads, HEAD_DIM)
    qp = jnp.dot(acc, projbd_ref[...], preferred_element_type=jnp.float32)
    bits = (qp > 0).astype(jnp.bfloat16)
    bk = jnp.dot(bits, powbd_ref[...], preferred_element_type=jnp.float32)
    bk_ref[...] = bk.astype(jnp.int32)[None]


def _q_proj(x2d, wq, bq, projbd, powbd, bm, bn):
    m_tiles = x2d.shape[0] // bm
    n_tiles = HIDDEN // bn
    heads_per_n = bn // HEAD_DIM
    return pl.pallas_call(
        _q_body,
        grid=(n_tiles, m_tiles),
        in_specs=[
            pl.BlockSpec((bm, HIDDEN), lambda n, m: (m, 0)),
            pl.BlockSpec((HIDDEN, bn), lambda n, m: (0, n)),
            pl.BlockSpec((1, bn), lambda n, m: (0, n)),
            pl.BlockSpec((bn, NUM_HASHES * heads_per_n), lambda n, m: (0, 0)),
            pl.BlockSpec((NUM_HASHES * heads_per_n, heads_per_n),
                         lambda n, m: (0, 0)),
        ],
        out_specs=[
            pl.BlockSpec((bm, heads_per_n, HEAD_DIM), lambda n, m: (m, n, 0)),
            pl.BlockSpec((1, bm, heads_per_n), lambda n, m: (n, m, 0)),
        ],
        out_shape=[
            jax.ShapeDtypeStruct((B * S, NUM_HEADS, HEAD_DIM), jnp.float32),
            jax.ShapeDtypeStruct((n_tiles, B * S, heads_per_n), jnp.int32),
        ],
    )(x2d, wq, bq.reshape(1, -1), projbd, powbd)


def _kv_body(x_ref, w_ref, b_ref, out_ref):
    acc = jnp.dot(x_ref[...], w_ref[...], preferred_element_type=jnp.float32)
    acc = acc + b_ref[...]
    bm = acc.shape[0]
    heads = acc.shape[1] // KV
    for hh in range(heads):
        kv = acc[:, hh * KV:(hh + 1) * KV].reshape(bm, 2, HEAD_DIM)
        inter = kv.reshape(2 * bm, HEAD_DIM).astype(jnp.bfloat16)
        out_ref[:, hh] = pltpu.bitcast(inter, jnp.int32)


def _kv_proj(xb, wkv, bkv, bm, bn):
    m_tiles = xb.shape[0] // bm
    n_tiles = wkv.shape[1] // bn
    heads_per_n = bn // KV
    return pl.pallas_call(
        _kv_body,
        grid=(n_tiles, m_tiles),
        in_specs=[
            pl.BlockSpec((bm, HIDDEN), lambda n, m: (m, 0)),
            pl.BlockSpec((HIDDEN, bn), lambda n, m: (0, n)),
            pl.BlockSpec((1, bn), lambda n, m: (0, n)),
        ],
        out_specs=pl.BlockSpec(
            (bm, heads_per_n, KVW), lambda n, m: (m, n, 0)),
        out_shape=jax.ShapeDtypeStruct((B * S, NUM_HEADS, KVW), jnp.int32),
    )(xb, wkv, bkv.reshape(1, -1))


def _sc_sort_gather(buckets_flat, q_table, kv_table):
    mesh = plsc.VectorSubcoreMesh(core_axis_name="c", subcore_axis_name="s")

    @functools.partial(
        pl.kernel,
        out_type=[
            jax.ShapeDtypeStruct((B * S * NUM_HEADS, HEAD_DIM), jnp.float32),
            jax.ShapeDtypeStruct((B * S * NUM_HEADS, KVW), jnp.int32),
        ],
        mesh=mesh,
        scratch_types=[
            pltpu.VMEM((S,), jnp.int32),
            pltpu.VMEM((16 * 256,), jnp.int32),
            pltpu.VMEM((S,), jnp.int32),
            pltpu.VMEM((N_CHUNKS, CHUNK), jnp.int32),
            pltpu.VMEM((CHUNK, HEAD_DIM), jnp.float32),
            pltpu.VMEM((CHUNK, HEAD_DIM), jnp.float32),
            pltpu.VMEM((CHUNK, KVW), jnp.int32),
            pltpu.VMEM((CHUNK, KVW), jnp.int32),
            pltpu.SMEM((256,), jnp.int32),
            pltpu.SemaphoreType.DMA,
            pltpu.SemaphoreType.DMA,
            pltpu.SemaphoreType.DMA,
            pltpu.SemaphoreType.DMA,
            pltpu.SemaphoreType.DMA,
            pltpu.SemaphoreType.DMA,
            pltpu.SemaphoreType.DMA,
            pltpu.SemaphoreType.DMA,
        ],
        compiler_params=pltpu.CompilerParams(needs_layout_passes=False),
    )
    def sc_kernel(buckets_hbm, q_hbm, kv_hbm, qs_hbm, kvs_hbm,
                  keys, hist2d, gidx, sidx, qr0, qr1, kvr0, kvr1, offs,
                  gq0, gq1, gkv0, gkv1, sq0, sq1, skv0, skv1):
        w = lax.axis_index("s") * 2 + lax.axis_index("c")
        b = w // NUM_HEADS
        h = w % NUM_HEADS
        base = b * S * NUM_HEADS + h

        lane = lax.iota(jnp.int32, 16)
        zero16 = jnp.zeros((16,), jnp.int32)
        ones16 = jnp.ones((16,), jnp.int32)

        pltpu.sync_copy(buckets_hbm.at[pl.ds(w * S, S)], keys)

        for j in range(16 * 256 // 16):
            hist2d[pl.ds(j * 16, 16)] = zero16

        lane256 = lane * 256
        def hist_body(c, carry):
            k16 = keys[pl.ds(c * 16, 16)]
            slot = lane256 + k16
            cnt = plsc.load_gather(hist2d, [slot])
            plsc.store_scatter(hist2d, [slot], cnt + ones16)
            return carry
        lax.fori_loop(0, S // 16, hist_body, 0, unroll=4)

        carry_in = jnp.int32(0)
        for g in range(16):
            tot = zero16
            for l in range(16):
                tot = tot + hist2d[pl.ds(l * 256 + g * 16, 16)]
            incl = plsc.cumsum(tot)
            excl = incl - tot + carry_in
            for l in range(16):
                offs[g * 16 + l] = excl[l]
            carry_in = carry_in + incl[15]

        def place_body(c, carry):
            k16 = keys[pl.ds(c * 16, 16)]
            src16 = base + (c * 16 + lane) * NUM_HEADS
            rvec = zero16
            for l in range(16):
                k = k16[l]
                r = offs[k]
                offs[k] = r + 1
                rvec = jnp.where(lane == l, r, rvec)
            plsc.store_scatter(gidx, [rvec], src16)
            return carry
        lax.fori_loop(0, S // 16, place_body, 0)

        for c in range(N_CHUNKS):
            for g in range(CHUNK // 16):
                t0 = c * CHUNK + g * 16
                sidx[c, pl.ds(g * 16, 16)] = (
                    base + (t0 + lane) * NUM_HEADS)

        qbufs = (qr0, qr1)
        kvbufs = (kvr0, kvr1)
        gqs = (gq0, gq1)
        gkvs = (gkv0, gkv1)
        sqs = (sq0, sq1)
        skvs = (skv0, skv1)

        def chunk_step(c, p):
            @pl.when(c >= 2)
            def _():
                pltpu.make_async_copy(qbufs[p], qs_hbm.at[sidx.at[c - 2]],
                                      sqs[p]).wait()
                pltpu.make_async_copy(kvbufs[p], kvs_hbm.at[sidx.at[c - 2]],
                                      skvs[p]).wait()

            gi = gidx.at[pl.ds(c * CHUNK, CHUNK)]
            pltpu.make_async_copy(q_hbm.at[gi], qbufs[p], gqs[p]).start()
            pltpu.make_async_copy(kv_hbm.at[gi], kvbufs[p], gkvs[p]).start()
            pltpu.make_async_copy(q_hbm.at[gi], qbufs[p], gqs[p]).wait()
            pltpu.make_async_copy(kv_hbm.at[gi], kvbufs[p], gkvs[p]).wait()
            pltpu.make_async_copy(qbufs[p], qs_hbm.at[sidx.at[c]],
                                  sqs[p]).start()
            pltpu.make_async_copy(kvbufs[p], kvs_hbm.at[sidx.at[c]],
                                  skvs[p]).start()

        def outer(c, carry):
            chunk_step(c * 2, 0)
            chunk_step(c * 2 + 1, 1)
            return carry
        lax.fori_loop(0, N_CHUNKS // 2, outer, 0)

        for p in range(2):
            c = N_CHUNKS - 2 + p
            pltpu.make_async_copy(qbufs[p], qs_hbm.at[sidx.at[c]],
                                  sqs[p]).wait()
            pltpu.make_async_copy(kvbufs[p], kvs_hbm.at[sidx.at[c]],
                                  skvs[p]).wait()

    return sc_kernel(buckets_flat, q_table, kv_table)


def _attn_body(q_ref, kv_ref, wo_ref, bo_ref, out_ref, att_ref,
               s_ref, p_ref):
    n_groups = q_ref.shape[0]
    R = GROUP * NUM_HEADS

    for g in range(n_groups):
        qg = q_ref[g].astype(jnp.bfloat16)
        kvi = pltpu.bitcast(kv_ref[g], jnp.bfloat16)
        s_ref[g * R:(g + 1) * R, :] = lax.dot_general(
            qg, kvi, (((1,), (1,)), ((), ())),
            preferred_element_type=jnp.float32)

    rows = n_groups * R
    r = lax.broadcasted_iota(jnp.int32, (rows, 2 * R), 0)
    c = lax.broadcasted_iota(jnp.int32, (rows, 2 * R), 1)
    keep = ((c % 2) == 0) & ((c // (2 * NUM_HEADS)) == (r % R) // NUM_HEADS)
    s = jnp.where(keep, s_ref[...] * SCALE, -jnp.inf)
    s = s - jnp.max(s, axis=-1, keepdims=True)
    e = jnp.exp(s)
    p = (e / jnp.sum(e, axis=-1, keepdims=True)).astype(jnp.bfloat16)
    p_ref[...] = jnp.roll(p, 1, axis=1)

    for g in range(n_groups):
        og = jnp.dot(p_ref[g * R:(g + 1) * R, :],
                     pltpu.bitcast(kv_ref[g], jnp.bfloat16),
                     preferred_element_type=jnp.float32)
        att_ref[g * GROUP:(g + 1) * GROUP, :] = og.astype(
            jnp.bfloat16).reshape(GROUP, HIDDEN)

    out_ref[...] = (
        jnp.dot(att_ref[...], wo_ref[...], preferred_element_type=jnp.float32)
        + bo_ref[...])


def _attn_proj(q_s, kv_s, wo_bf16, bo, bm):
    m_tiles = (B * S) // bm
    g = bm // GROUP
    return pl.pallas_call(
        _attn_body,
        grid=(m_tiles,),
        in_specs=[
            pl.BlockSpec((g, GROUP * NUM_HEADS, HEAD_DIM),
                         lambda m: (m, 0, 0)),
            pl.BlockSpec((g, GROUP * NUM_HEADS, KVW), lambda m: (m, 0, 0)),
            pl.BlockSpec((HIDDEN, HIDDEN), lambda m: (0, 0)),
            pl.BlockSpec((1, HIDDEN), lambda m: (0, 0)),
        ],
        out_specs=pl.BlockSpec((bm, HIDDEN), lambda m: (m, 0)),
        out_shape=jax.ShapeDtypeStruct((B * S, HIDDEN), jnp.float32),
        scratch_shapes=[
            pltpu.VMEM((bm, HIDDEN), jnp.bfloat16),
            pltpu.VMEM((g * GROUP * NUM_HEADS, 2 * GROUP * NUM_HEADS),
                       jnp.float32),
            pltpu.VMEM((g * GROUP * NUM_HEADS, 2 * GROUP * NUM_HEADS),
                       jnp.bfloat16),
        ],
    )(q_s, kv_s, wo_bf16, bo.reshape(1, -1))


@jax.jit
def kernel(x, Wq, bq, Wk, bk, Wv, bv, Wo, bo, proj):
    x2d = x.reshape(B * S, HIDDEN)
    xb = x2d.astype(jnp.bfloat16)
    wkv = jnp.concatenate(
        [Wk.reshape(HIDDEN, NUM_HEADS, 1, HEAD_DIM),
         Wv.reshape(HIDDEN, NUM_HEADS, 1, HEAD_DIM)], axis=2,
    ).reshape(HIDDEN, 2 * HIDDEN).astype(jnp.bfloat16)
    bkv = jnp.concatenate(
        [bk.reshape(NUM_HEADS, 1, HEAD_DIM), bv.reshape(NUM_HEADS, 1,
                                                        HEAD_DIM)], axis=1,
    ).reshape(2 * HIDDEN)

    heads_per_n = 1024 // HEAD_DIM
    projbd = jnp.kron(jnp.eye(heads_per_n, dtype=jnp.float32), proj.T)
    powers = (2.0 ** jnp.arange(NUM_HASHES, dtype=jnp.float32))[:, None]
    powbd = jnp.kron(jnp.eye(heads_per_n, dtype=jnp.float32),
                     powers).astype(jnp.bfloat16)

    q_tab, bk = _q_proj(x2d, Wq, bq, projbd, powbd, bm=512, bn=1024)
    kv_tab = _kv_proj(xb, wkv, bkv, bm=512, bn=2048)

    buckets = bk.reshape(2, B, S, NUM_HEADS // 2).transpose(
        (1, 0, 3, 2)).reshape(-1)

    q_s, kv_s = _sc_sort_gather(
        buckets,
        q_tab.reshape(B * NUM_HEADS * S, HEAD_DIM),
        kv_tab.reshape(B * NUM_HEADS * S, KVW))
    q_s = q_s.reshape(B * S // GROUP, GROUP * NUM_HEADS, HEAD_DIM)
    kv_s = kv_s.reshape(B * S // GROUP, GROUP * NUM_HEADS, KVW)

    out = _attn_proj(q_s, kv_s, Wo.astype(jnp.bfloat16), bo, bm=256)
    return out.reshape(B, S, HIDDEN)
